# Initial kernel scaffold; baseline (speedup 1.0000x reference)
import jax
import jax.numpy as jnp
from jax.experimental import pallas as pl


def kernel(x, assign0, assign1, assign2, V0, g0, V1, g1, V2, g2, fc_V, fc_g, fc_b):
    raise NotImplementedError("write your pallas kernel here")


# speedup vs baseline: 116.4434x; 116.4434x over previous
import jax
import jax.numpy as jnp
from jax import lax
from jax.experimental import pallas as pl
from jax.experimental.pallas import tpu as pltpu
from jax.experimental.pallas import tpu_sc as plsc

B = 16
N0 = 100000
K = 128
NCLS = 10
NW = 32
CHUNK = 3200
NPAD_SC = NW * CHUNK
BLK = 2048
NBLK = 49
NPAD_TC = NBLK * BLK


def _fa_body(a0_hbm, a1_hbm, a2_hbm, fa_hbm, i0_v, i1_v, i2_v):
    c = lax.axis_index("core")
    s = lax.axis_index("subcore")
    wid = s * 2 + c
    base = wid * CHUNK
    pltpu.sync_copy(a0_hbm.at[pl.ds(base, CHUNK)], i0_v)
    pltpu.sync_copy(a1_hbm.at[i0_v], i1_v)
    pltpu.sync_copy(a2_hbm.at[i1_v], i2_v)
    pltpu.sync_copy(i2_v, fa_hbm.at[pl.ds(base, CHUNK)])


def _segsum_body(x_ref, fa_ref, out_ref):
    i = pl.program_id(0)

    @pl.when(i == 0)
    def _init():
        out_ref[...] = jnp.zeros_like(out_ref)

    col = i * BLK + lax.broadcasted_iota(jnp.int32, (B, BLK), 1)
    xb = jnp.where(col < N0, x_ref[...], 0.0)
    fa = fa_ref[0]
    onehot_t = (jnp.broadcast_to(fa, (K, BLK)) ==
                lax.broadcasted_iota(jnp.int32, (K, BLK), 0)
                ).astype(jnp.float32)
    out_ref[...] += lax.dot_general(
        xb, onehot_t, (((1,), (1,)), ((), ())),
        preferred_element_type=jnp.float32)


def _epi_body(s2_ref, v0_ref, g0_ref, v1_ref, g1_ref, v2_ref, g2_ref,
              m_ref, fcg_ref, fcb_ref, out_ref):
    def wn(v, g):
        n = jnp.sqrt(jnp.sum(v * v, axis=1, keepdims=True))
        return g * v / (n + 1e-12)

    def mm(a, b):
        return lax.dot_general(a, b, (((1,), (0,)), ((), ())),
                               preferred_element_type=jnp.float32)

    w0 = wn(v0_ref[...], g0_ref[...])
    w1 = wn(v1_ref[...], g1_ref[...])
    w2 = wn(v2_ref[...], g2_ref[...])
    w = mm(w2, mm(w1, w0))
    wrep = jnp.concatenate([w] * NCLS, axis=0)
    r0 = lax.broadcasted_iota(jnp.int32, (NCLS * K, NCLS), 0)
    r1 = lax.broadcasted_iota(jnp.int32, (NCLS * K, NCLS), 1)
    sel = (jnp.right_shift(r0, 7) == r1).astype(jnp.float32)
    m = m_ref[...]
    d = lax.dot_general(s2_ref[...], m, (((1,), (1,)), ((), ())),
                        preferred_element_type=jnp.float32)
    e = mm(d, sel * wrep)
    rowsq = jnp.sum(m * m, axis=1, keepdims=True)
    nsq = mm(jnp.ones((1, NCLS * K), jnp.float32), sel * rowsq)
    scale = fcg_ref[...] / (jnp.sqrt(nsq) + 1e-12)
    out_ref[...] = scale * e + fcb_ref[...]


def kernel(x, assign0, assign1, assign2, V0, g0, V1, g1, V2, g2,
           fc_V, fc_g, fc_b):
    a0p = jnp.concatenate(
        [assign0, jnp.zeros((NPAD_SC - N0,), jnp.int32)])

    vector_mesh = plsc.VectorSubcoreMesh(
        core_axis_name="core", subcore_axis_name="subcore")
    fa = pl.kernel(
        out_type=jax.ShapeDtypeStruct((NPAD_SC,), jnp.int32),
        mesh=vector_mesh,
        scratch_types=[pltpu.VMEM((CHUNK,), jnp.int32)] * 3,
    )(_fa_body)(a0p, assign1, assign2)

    fa3 = fa[:NPAD_TC].reshape(NBLK, 1, BLK)
    s2 = pl.pallas_call(
        _segsum_body,
        grid=(NBLK,),
        in_specs=[pl.BlockSpec((B, BLK), lambda i: (0, i)),
                  pl.BlockSpec((1, 1, BLK), lambda i: (i, 0, 0))],
        out_specs=pl.BlockSpec((B, K), lambda i: (0, 0)),
        out_shape=jax.ShapeDtypeStruct((B, K), jnp.float32),
    )(x, fa3)

    out = pl.pallas_call(
        _epi_body,
        out_shape=jax.ShapeDtypeStruct((B, NCLS), jnp.float32),
    )(s2, V0, g0.reshape(32, 1), V1, g1.reshape(64, 1), V2,
      g2.reshape(128, 1), fc_V.reshape(NCLS * K, K),
      fc_g.reshape(1, NCLS), fc_b.reshape(1, NCLS))
    return out


# speedup vs baseline: 213.2108x; 1.8310x over previous
import dataclasses

import jax
import jax.numpy as jnp
from jax import lax
from jax.experimental import pallas as pl
from jax.experimental.pallas import tpu as pltpu
from jax.experimental.pallas import tpu_sc as plsc

B = 16
N0 = 100000
N1 = 65536
N2 = 16384
K = 128
NCLS = 10
NW = 32
CHUNK = 3136
LAST_BASE = N0 - CHUNK
BLK = 7168
NBLK = 14
NPAD_TC = NBLK * BLK


def _fa_body(a0_hbm, a1_hbm, a2_hbm, fa_hbm, t1_v, t2_v, i0_v, i2_v, sems):
    c = lax.axis_index("core")
    s = lax.axis_index("subcore")
    wid = s * 2 + c
    base = lax.min(wid * CHUNK, LAST_BASE)
    cp1 = pltpu.async_copy(a1_hbm, t1_v, sems.at[0])
    cp2 = pltpu.async_copy(a2_hbm, t2_v, sems.at[1])
    cp3 = pltpu.async_copy(a0_hbm.at[pl.ds(base, CHUNK)], i0_v, sems.at[2])
    cp1.wait()
    cp2.wait()
    cp3.wait()

    @pl.loop(0, CHUNK // 16)
    def _(v):
        i0 = i0_v[pl.ds(v * 16, 16)]
        i1 = plsc.load_gather(t1_v, [i0])
        i2 = plsc.load_gather(t2_v, [i1])
        i2_v[pl.ds(v * 16, 16)] = i2

    pltpu.sync_copy(i2_v, fa_hbm.at[pl.ds(base, CHUNK)])


def _epilogue(s2, v0_ref, g0_ref, v1_ref, g1_ref, v2_ref, g2_ref,
              m_ref, fcg_ref, fcb_ref, out_ref):
    def wn(v, g):
        n = jnp.sqrt(jnp.sum(v * v, axis=1, keepdims=True))
        return g * v / (n + 1e-12)

    def mm(a, b):
        return lax.dot_general(a, b, (((1,), (0,)), ((), ())),
                               preferred_element_type=jnp.float32)

    w0 = wn(v0_ref[...], g0_ref[...])
    w1 = wn(v1_ref[...], g1_ref[...])
    w2 = wn(v2_ref[...], g2_ref[...])
    w = mm(w2, mm(w1, w0))
    wrep = jnp.concatenate([w] * NCLS, axis=0)
    r0 = lax.broadcasted_iota(jnp.int32, (NCLS * K, NCLS), 0)
    r1 = lax.broadcasted_iota(jnp.int32, (NCLS * K, NCLS), 1)
    sel = (jnp.right_shift(r0, 7) == r1).astype(jnp.float32)
    m = m_ref[...]
    d = lax.dot_general(s2, m, (((1,), (1,)), ((), ())),
                        preferred_element_type=jnp.float32)
    e = mm(d, sel * wrep)
    rowsq = jnp.sum(m * m, axis=1, keepdims=True)
    nsq = mm(jnp.ones((1, NCLS * K), jnp.float32), sel * rowsq)
    scale = fcg_ref[...] / (jnp.sqrt(nsq) + 1e-12)
    out_ref[...] = scale * e + fcb_ref[...]


def _main_body(x_ref, fa_ref, v0_ref, g0_ref, v1_ref, g1_ref, v2_ref, g2_ref,
               m_ref, fcg_ref, fcb_ref, out_ref, s2_ref):
    i = pl.program_id(0)

    @pl.when(i == 0)
    def _init():
        s2_ref[...] = jnp.zeros_like(s2_ref)

    col = i * BLK + lax.broadcasted_iota(jnp.int32, (B, BLK), 1)
    xb = jnp.where(col < N0, x_ref[...], 0.0).astype(jnp.bfloat16)
    fa = fa_ref[0]
    onehot_t = (jnp.broadcast_to(fa, (K, BLK)) ==
                lax.broadcasted_iota(jnp.int32, (K, BLK), 0)
                ).astype(jnp.bfloat16)
    s2_ref[...] += lax.dot_general(
        xb, onehot_t, (((1,), (1,)), ((), ())),
        preferred_element_type=jnp.float32)

    @pl.when(i == NBLK - 1)
    def _fin():
        _epilogue(s2_ref[...], v0_ref, g0_ref, v1_ref, g1_ref, v2_ref,
                  g2_ref, m_ref, fcg_ref, fcb_ref, out_ref)


def kernel(x, assign0, assign1, assign2, V0, g0, V1, g1, V2, g2,
           fc_V, fc_g, fc_b):
    vector_mesh = plsc.VectorSubcoreMesh(
        core_axis_name="core", subcore_axis_name="subcore")
    sc_params = pltpu.CompilerParams()
    if "needs_layout_passes" in pltpu.CompilerParams.__dataclass_fields__:
        sc_params = dataclasses.replace(sc_params, needs_layout_passes=False)
    fa = pl.kernel(
        out_type=jax.ShapeDtypeStruct((NPAD_TC,), jnp.int32),
        mesh=vector_mesh,
        scratch_types=[pltpu.VMEM((N1,), jnp.int32),
                       pltpu.VMEM((N2,), jnp.int32),
                       pltpu.VMEM((CHUNK,), jnp.int32),
                       pltpu.VMEM((CHUNK,), jnp.int32),
                       pltpu.SemaphoreType.DMA((3,))],
        compiler_params=sc_params,
    )(_fa_body)(assign0, assign1, assign2)

    fa3 = fa.reshape(NBLK, 1, BLK)
    const = lambda i: (0, 0)
    out = pl.pallas_call(
        _main_body,
        grid=(NBLK,),
        in_specs=[pl.BlockSpec((B, BLK), lambda i: (0, i)),
                  pl.BlockSpec((1, 1, BLK), lambda i: (i, 0, 0)),
                  pl.BlockSpec((32, 1), const),
                  pl.BlockSpec((32, 1), const),
                  pl.BlockSpec((64, 32), const),
                  pl.BlockSpec((64, 1), const),
                  pl.BlockSpec((128, 64), const),
                  pl.BlockSpec((128, 1), const),
                  pl.BlockSpec((NCLS * K, K), const),
                  pl.BlockSpec((1, NCLS), const),
                  pl.BlockSpec((1, NCLS), const)],
        out_specs=pl.BlockSpec((B, NCLS), const),
        out_shape=jax.ShapeDtypeStruct((B, NCLS), jnp.float32),
        scratch_shapes=[pltpu.VMEM((B, K), jnp.float32)],
    )(x, fa3, V0, g0.reshape(32, 1), V1, g1.reshape(64, 1), V2,
      g2.reshape(128, 1), fc_V.reshape(NCLS * K, K),
      fc_g.reshape(1, NCLS), fc_b.reshape(1, NCLS))
    return out
